# baseline (speedup 1.0000x reference)
import functools

import jax
import jax.numpy as jnp
from jax import lax
from jax.experimental import pallas as pl
from jax.experimental.pallas import tpu as pltpu
from jax.experimental.pallas import tpu_sc as plsc

_NC = 2
_NS = 16
_L = 16
_SB = 16
_CH = 256


def _tc_argmax(B, N, F):
    assert B % _SB == 0 and N % _CH == 0 and F % 128 == 0
    cblk = F // 128 - 1
    nc = N // _CH
    nv = _CH // 8

    def body(x_ref, out_ref):
        gb = pl.program_id(0)
        sub = lax.broadcasted_iota(jnp.int32, (8, 128), 0)
        lane127 = lax.broadcasted_iota(jnp.int32, (8, 128), 1) == 127
        big = jnp.full((8, 128), jnp.int32(1 << 30))

        def pairmax(a, b):
            gt = b[0] > a[0]
            return jnp.where(gt, b[0], a[0]), jnp.where(gt, b[1], a[1])

        for i in range(_SB):
            chunks = []
            for c in range(nc):
                y = x_ref[i, pl.ds(c * _CH, _CH), :].reshape(nv, 8, 128)
                nodes = []
                for k in range(nv // 2):
                    g0 = jnp.int32(c * nv + 2 * k)
                    g1 = jnp.int32(c * nv + 2 * k + 1)
                    gt = y[2 * k + 1] > y[2 * k]
                    nodes.append((jnp.where(gt, y[2 * k + 1], y[2 * k]),
                                  jnp.where(gt, g1, g0)))
                while len(nodes) > 1:
                    nodes = [pairmax(nodes[k], nodes[k + 1])
                             for k in range(0, len(nodes), 2)]
                chunks.append(nodes[0])
            while len(chunks) > 1:
                chunks = [pairmax(chunks[k], chunks[k + 1])
                          for k in range(0, len(chunks), 2)]
            rm, ri = chunks[0]

            m = jnp.max(jnp.where(lane127, rm, -jnp.inf))
            cand = jnp.where((rm == m) & lane127, ri * 8 + sub, big)
            n = jnp.min(cand)
            row = (gb * _SB + i) * N + n
            out_ref[i, :] = row + jnp.zeros((128,), jnp.int32)

    return pl.pallas_call(
        body,
        grid=(B // _SB,),
        in_specs=[
            pl.BlockSpec((_SB, N, 128), lambda g: (g, 0, cblk)),
        ],
        out_specs=pl.BlockSpec((_SB, 128), lambda g: (g, 0)),
        out_shape=jax.ShapeDtypeStruct((B, 128), jnp.int32),
    )


def _sc_gather(B, N, F):
    bpw = 16
    nw = B // bpw
    mesh = plsc.VectorSubcoreMesh(core_axis_name="c", subcore_axis_name="s")

    @functools.partial(
        pl.kernel,
        mesh=mesh,
        out_type=jax.ShapeDtypeStruct((B, F), jnp.float32),
        compiler_params=pltpu.CompilerParams(needs_layout_passes=False),
        scratch_types=[
            pltpu.VMEM((bpw, 128), jnp.int32),
            pltpu.VMEM((bpw, F), jnp.float32),
            pltpu.SemaphoreType.DMA,
        ],
    )
    def sc_kernel(in2d, idx_hbm, out, idxbuf, rows_v, sem):
        wid = lax.axis_index("s") * _NC + lax.axis_index("c")

        @pl.when(wid < nw)
        def _():
            pltpu.sync_copy(idx_hbm.at[pl.ds(wid * bpw, bpw), :], idxbuf)
            lanes = lax.iota(jnp.int32, _L)
            rows = plsc.load_gather(idxbuf, [lanes, jnp.zeros((_L,), jnp.int32)])
            pltpu.async_copy(in2d.at[rows], rows_v, sem).wait()
            pltpu.sync_copy(rows_v, out.at[pl.ds(wid * bpw, bpw)])

    return sc_kernel


def kernel(inputs):
    B, N, F = inputs.shape
    in2d = inputs.reshape(B * N, F)
    idx = _tc_argmax(B, N, F)(inputs)
    return _sc_gather(B, N, F)(in2d, idx)


# speedup vs baseline: 1.0993x; 1.0993x over previous
import functools

import jax
import jax.numpy as jnp
from jax import lax
from jax.experimental import pallas as pl
from jax.experimental.pallas import tpu as pltpu
from jax.experimental.pallas import tpu_sc as plsc

_NC = 2
_NS = 16
_L = 16
_SB = 16
_CH = 256


def _tc_scan(B, N, F):
    assert B % _SB == 0 and N % _CH == 0 and F % 128 == 0
    cblk = F // 128 - 1
    nc = N // _CH
    nv = _CH // 8

    def body(x_ref, rm_ref, ri_ref):
        def pairmax(a, b):
            gt = b[0] > a[0]
            return jnp.where(gt, b[0], a[0]), jnp.where(gt, b[1], a[1])

        for i in range(_SB):
            chunks = []
            for c in range(nc):
                y = x_ref[i, pl.ds(c * _CH, _CH), :].reshape(nv, 8, 128)
                nodes = []
                for k in range(nv // 2):
                    g0 = jnp.int32(c * nv + 2 * k)
                    g1 = jnp.int32(c * nv + 2 * k + 1)
                    gt = y[2 * k + 1] > y[2 * k]
                    nodes.append((jnp.where(gt, y[2 * k + 1], y[2 * k]),
                                  jnp.where(gt, g1, g0)))
                while len(nodes) > 1:
                    nodes = [pairmax(nodes[k], nodes[k + 1])
                             for k in range(0, len(nodes), 2)]
                chunks.append(nodes[0])
            while len(chunks) > 1:
                chunks = [pairmax(chunks[k], chunks[k + 1])
                          for k in range(0, len(chunks), 2)]
            rm, ri = chunks[0]
            rm_ref[pl.ds(i, 1)] = rm.reshape(1, 8, 128)
            ri_ref[pl.ds(i, 1)] = ri.reshape(1, 8, 128)

    return pl.pallas_call(
        body,
        grid=(B // _SB,),
        in_specs=[
            pl.BlockSpec((_SB, N, 128), lambda g: (g, 0, cblk)),
        ],
        out_specs=[
            pl.BlockSpec((_SB, 8, 128), lambda g: (g, 0, 0)),
            pl.BlockSpec((_SB, 8, 128), lambda g: (g, 0, 0)),
        ],
        out_shape=[
            jax.ShapeDtypeStruct((B, 8, 128), jnp.float32),
            jax.ShapeDtypeStruct((B, 8, 128), jnp.int32),
        ],
    )


def _sc_finish(B, N, F):
    bpw = 8
    nw = B // bpw
    mesh = plsc.VectorSubcoreMesh(core_axis_name="c", subcore_axis_name="s")

    @functools.partial(
        pl.kernel,
        mesh=mesh,
        out_type=jax.ShapeDtypeStruct((B, F), jnp.float32),
        compiler_params=pltpu.CompilerParams(needs_layout_passes=False),
        scratch_types=[
            pltpu.VMEM((bpw, 8, 128), jnp.float32),
            pltpu.VMEM((bpw, 8, 128), jnp.int32),
            pltpu.VMEM((bpw,), jnp.int32),
            pltpu.VMEM((bpw, F), jnp.float32),
            pltpu.SemaphoreType.DMA,
        ],
    )
    def sc_kernel(in2d, rm_hbm, ri_hbm, out, rmb, rib, idx_ref, rows_v, sem):
        wid = lax.axis_index("s") * _NC + lax.axis_index("c")

        @pl.when(wid < nw)
        def _():
            b0 = wid * bpw
            pltpu.sync_copy(rm_hbm.at[pl.ds(b0, bpw)], rmb)
            pltpu.sync_copy(ri_hbm.at[pl.ds(b0, bpw)], rib)
            lanes = lax.iota(jnp.int32, _L)
            sv = lanes & 7
            c127 = jnp.full((_L,), 127, jnp.int32)
            lo = lanes < 8
            neg_inf = jnp.full((_L,), -jnp.inf, jnp.float32)
            big = jnp.full((_L,), jnp.int32(1 << 30))
            idxvec = jnp.zeros((_L,), jnp.int32)

            for p in range(bpw // 2):
                q = 2 * p + (lanes >> 3)
                v = plsc.load_gather(rmb, [q, sv, c127])
                iv = plsc.load_gather(rib, [q, sv, c127])
                rowv = (b0 + q) * N + iv * 8 + sv
                m0 = jnp.max(jnp.where(lo, v, neg_inf))
                n0 = jnp.min(jnp.where((v == m0) & lo, rowv, big))
                m1 = jnp.max(jnp.where(lo, neg_inf, v))
                n1 = jnp.min(jnp.where((v == m1) & ~lo, rowv, big))
                idxvec = jnp.where(lanes == 2 * p, n0, idxvec)
                idxvec = jnp.where(lanes == 2 * p + 1, n1, idxvec)

            plsc.store_scatter(idx_ref, [sv], idxvec, mask=lo)
            pltpu.async_copy(in2d.at[idx_ref], rows_v, sem).wait()
            pltpu.sync_copy(rows_v, out.at[pl.ds(b0, bpw)])

    return sc_kernel


def kernel(inputs):
    B, N, F = inputs.shape
    in2d = inputs.reshape(B * N, F)
    rm, ri = _tc_scan(B, N, F)(inputs)
    return _sc_finish(B, N, F)(in2d, rm, ri)
